# baseline (speedup 1.0000x reference)
import jax
import jax.numpy as jnp
from jax import lax
from jax.experimental import pallas as pl
from jax.experimental.pallas import tpu as pltpu
from jax.experimental.pallas import tpu_sc as plsc

NC = 2
NS = 16
NW = NC * NS
H = 200
BT = 16384
V = 49
D = 64
DP = 65
TR = 8
SUB = 4096
TPB = SUB // 128
NQ = BT // SUB
UNITS = (H * TR) // NW


def _tec_body(idxT_hbm, tbl_hbm, out_hbm, tbl_v, idx_v, out_v, sw0, sw1):
    wid = lax.axis_index("s") * NC + lax.axis_index("c")
    pltpu.sync_copy(tbl_hbm, tbl_v)
    sw = (sw0, sw1)

    def drain(p):
        pltpu.make_async_copy(
            out_v.at[p], out_hbm.at[0, 0, pl.ds(0, TPB)], sw[p]
        ).wait()

    u0 = wid * UNITS

    def unit_body(u, h_prev):
        uu = u0 + u
        h = uu // TR
        tr = uu % TR

        @pl.when(h != h_prev)
        def _():
            pltpu.sync_copy(idxT_hbm.at[h], idx_v)

        for q in range(NQ):
            p = q % 2

            @pl.when(jnp.logical_or(u > 0, q >= 2))
            def _():
                drain(p)

            @plsc.parallel_loop(0, TPB, unroll=3)
            def jbody(j):
                for bb in range(8):
                    idx16 = idx_v[pl.ds(q * SUB + (j * 8 + bb) * 16, 16)]
                    base = idx16 * DP + tr * 8
                    for ci in range(8):
                        val = plsc.load_gather(tbl_v, [base + ci])
                        out_v[p, j, ci, pl.ds(bb * 16, 16)] = val
            pltpu.async_copy(
                out_v.at[p], out_hbm.at[h, tr, pl.ds(q * TPB, TPB)], sw[p]
            )
        return h

    lax.fori_loop(0, UNITS, unit_body, -1)
    drain(0)
    drain(1)


def kernel(time, table):
    BATCH, HIST = time.shape
    idxT = time.astype(jnp.int32).T
    tbl_flat = jnp.pad(table, ((0, 0), (0, DP - D))).reshape(-1)

    mesh = plsc.VectorSubcoreMesh(core_axis_name="c", subcore_axis_name="s")
    run = pl.kernel(
        _tec_body,
        out_type=jax.ShapeDtypeStruct((H, TR, BT // 128, D // TR, 128), jnp.float32),
        mesh=mesh,
        scratch_types=[
            pltpu.VMEM((V * DP,), jnp.float32),
            pltpu.VMEM((BT,), jnp.int32),
            pltpu.VMEM((2, TPB, D // TR, 128), jnp.float32),
            pltpu.SemaphoreType.DMA,
            pltpu.SemaphoreType.DMA,
        ],
        compiler_params=pltpu.CompilerParams(
            use_tc_tiling_on_sc=False, needs_layout_passes=False
        ),
    )
    out5 = run(idxT, tbl_flat)
    return out5.transpose(2, 4, 0, 1, 3).reshape(BATCH, HIST, D)


# speedup vs baseline: 1.1034x; 1.1034x over previous
import jax
import jax.numpy as jnp
from jax import lax
from jax.experimental import pallas as pl
from jax.experimental.pallas import tpu as pltpu
from jax.experimental.pallas import tpu_sc as plsc

NC = 2
NS = 16
NW = NC * NS
H = 200
BT = 16384
V = 49
D = 64
DP = 65
TR = 8
SUB = 4096
TPB = SUB // 128
NQ = BT // SUB
UNITS = (H * TR) // NW


def _tec_body(idxT_hbm, tbl_hbm, out_hbm, tbl_v, idx_v, out_v, sw0, sw1):
    wid = lax.axis_index("s") * NC + lax.axis_index("c")
    pltpu.sync_copy(tbl_hbm, tbl_v)
    sw = (sw0, sw1)

    def drain(p):
        pltpu.make_async_copy(
            out_v.at[p], out_hbm.at[0, 0, pl.ds(0, TPB)], sw[p]
        ).wait()

    u0 = wid * UNITS

    def unit_body(u, h_prev):
        uu = u0 + u
        h = uu // TR
        tr = uu % TR

        @pl.when(h != h_prev)
        def _():
            pltpu.sync_copy(idxT_hbm.at[h], idx_v)

        for q in range(NQ):
            p = q % 2

            @pl.when(jnp.logical_or(u > 0, q >= 2))
            def _():
                drain(p)

            @plsc.parallel_loop(0, TPB, unroll=2)
            def jbody(j):
                for bb in range(8):
                    idx16 = idx_v[pl.ds(q * SUB + (j * 8 + bb) * 16, 16)]
                    base = idx16 * DP + tr * 8
                    for ci in range(8):
                        val = plsc.load_gather(tbl_v, [base + ci])
                        out_v[p, j, ci, pl.ds(bb * 16, 16)] = val
            pltpu.async_copy(
                out_v.at[p], out_hbm.at[h, tr, pl.ds(q * TPB, TPB)], sw[p]
            )
        return h

    lax.fori_loop(0, UNITS, unit_body, -1)
    drain(0)
    drain(1)


def kernel(time, table):
    BATCH, HIST = time.shape
    idxT = time.astype(jnp.int32).T
    tbl_flat = jnp.pad(table, ((0, 0), (0, DP - D))).reshape(-1)

    mesh = plsc.VectorSubcoreMesh(core_axis_name="c", subcore_axis_name="s")
    run = pl.kernel(
        _tec_body,
        out_type=jax.ShapeDtypeStruct((H, TR, BT // 128, D // TR, 128), jnp.float32),
        mesh=mesh,
        scratch_types=[
            pltpu.VMEM((V * DP,), jnp.float32),
            pltpu.VMEM((BT,), jnp.int32),
            pltpu.VMEM((2, TPB, D // TR, 128), jnp.float32),
            pltpu.SemaphoreType.DMA,
            pltpu.SemaphoreType.DMA,
        ],
        compiler_params=pltpu.CompilerParams(
            use_tc_tiling_on_sc=False, needs_layout_passes=False
        ),
    )
    out5 = run(idxT, tbl_flat)
    return out5.transpose(2, 4, 0, 1, 3).reshape(BATCH, HIST, D)


# speedup vs baseline: 1.1552x; 1.0469x over previous
import jax
import jax.numpy as jnp
from jax import lax
from jax.experimental import pallas as pl
from jax.experimental.pallas import tpu as pltpu
from jax.experimental.pallas import tpu_sc as plsc

NC = 2
NS = 16
NW = NC * NS
H = 200
BT = 16384
V = 49
D = 64
DP = 65
TR = 8
SUB = 4096
TPB = SUB // 128
NQ = BT // SUB
UNITS = (H * TR) // NW


def _tec_body(idxT_hbm, tbl_hbm, out_hbm, tbl_v, idx_v, out_v, sw0, sw1, sidx):
    wid = lax.axis_index("s") * NC + lax.axis_index("c")
    pltpu.sync_copy(tbl_hbm, tbl_v)
    sw = (sw0, sw1)

    def fire_idx(hrow):
        hn = jnp.minimum(hrow, H - 1)
        pltpu.async_copy(idxT_hbm.at[hn], idx_v.at[hn % 2], sidx)

    def wait_idx():
        pltpu.make_async_copy(idxT_hbm.at[0], idx_v.at[0], sidx).wait()

    def drain(p):
        pltpu.make_async_copy(
            out_v.at[p], out_hbm.at[0, 0, pl.ds(0, TPB)], sw[p]
        ).wait()

    u0 = wid * UNITS
    h0 = u0 // TR
    pltpu.sync_copy(idxT_hbm.at[h0], idx_v.at[h0 % 2])
    fire_idx(h0 + 1)

    def unit_body(u, h_prev):
        uu = u0 + u
        h = uu // TR
        tr = uu % TR
        hb = h % 2

        @pl.when(h != h_prev)
        def _():
            wait_idx()
            fire_idx(h + 1)

        for q in range(NQ):
            p = q % 2

            @pl.when(jnp.logical_or(u > 0, q >= 2))
            def _():
                drain(p)

            @plsc.parallel_loop(0, TPB, unroll=2)
            def jbody(j):
                for bb in range(8):
                    idx16 = idx_v[hb, pl.ds(q * SUB + (j * 8 + bb) * 16, 16)]
                    base = idx16 * DP + tr * 8
                    for ci in range(8):
                        val = plsc.load_gather(tbl_v, [base + ci])
                        out_v[p, j, ci, pl.ds(bb * 16, 16)] = val
            pltpu.async_copy(
                out_v.at[p], out_hbm.at[h, tr, pl.ds(q * TPB, TPB)], sw[p]
            )
        return h

    lax.fori_loop(0, UNITS, unit_body, h0)
    wait_idx()
    drain(0)
    drain(1)


def kernel(time, table):
    BATCH, HIST = time.shape
    idxT = time.astype(jnp.int32).T
    tbl_flat = jnp.pad(table, ((0, 0), (0, DP - D))).reshape(-1)

    mesh = plsc.VectorSubcoreMesh(core_axis_name="c", subcore_axis_name="s")
    run = pl.kernel(
        _tec_body,
        out_type=jax.ShapeDtypeStruct((H, TR, BT // 128, D // TR, 128), jnp.float32),
        mesh=mesh,
        scratch_types=[
            pltpu.VMEM((V * DP,), jnp.float32),
            pltpu.VMEM((2, BT), jnp.int32),
            pltpu.VMEM((2, TPB, D // TR, 128), jnp.float32),
            pltpu.SemaphoreType.DMA,
            pltpu.SemaphoreType.DMA,
            pltpu.SemaphoreType.DMA,
        ],
        compiler_params=pltpu.CompilerParams(
            use_tc_tiling_on_sc=False, needs_layout_passes=False
        ),
    )
    out5 = run(idxT, tbl_flat)
    return out5.transpose(2, 4, 0, 1, 3).reshape(BATCH, HIST, D)
